# baseline (speedup 1.0000x reference)
import functools

import jax
import jax.numpy as jnp
from jax import lax
from jax.experimental import pallas as pl
from jax.experimental.pallas import tpu as pltpu
from jax.experimental.pallas import tpu_sc as plsc

_NMUT = 5
_MAXCOS = 0.99
_LANES = 16
_GRAN = 128
_KB = 2048


def _build_sc_gather(B, L, C, d):
    npair = _GRAN // d
    pshift = npair.bit_length() - 1
    assert npair == (1 << pshift) and d * npair == _GRAN and L % npair == 0

    def body(wdp_hbm, seqf_hbm, pos_hbm, nuc_hbm,
             outn_hbm, outo_hbm, outi_hbm, outj_hbm,
             pos_v, nuc_v, idx_old, idx_new, idxr1, idxr2,
             idx1, idx2, idx3, g1, g2, g3,
             wold_v, wnew_v, sem_g, sem_w, sem_io):
        cid = lax.axis_index("c")
        sid = lax.axis_index("s")
        wid = sid * 2 + cid
        cp_p = pltpu.async_copy(pos_hbm.at[wid], pos_v, sem_io)
        cp_n = pltpu.async_copy(nuc_hbm.at[wid], nuc_v, sem_io)
        cp_p.wait()
        cp_n.wait()
        posv = pos_v[...]
        nucv = nuc_v[...]
        idx_new[...] = nucv * L + posv
        idxr1[...] = lax.shift_right_logical(nucv * L + posv, pshift)
        cp_new = pltpu.async_copy(wdp_hbm.at[idxr1], wnew_v, sem_w)
        base = jnp.broadcast_to(wid * L, (_LANES,)).astype(jnp.int32) + posv
        idx1[...] = base + (B * L)
        cp1 = pltpu.async_copy(seqf_hbm.at[idx1], g1, sem_g)
        idx2[...] = base + 2 * (B * L)
        cp2 = pltpu.async_copy(seqf_hbm.at[idx2], g2, sem_g)
        idx3[...] = base + 3 * (B * L)
        cp3 = pltpu.async_copy(seqf_hbm.at[idx3], g3, sem_g)
        cp1.wait()
        cp2.wait()
        cp3.wait()
        c_old = (g1[...] + 2.0 * g2[...] + 3.0 * g3[...]).astype(jnp.int32)
        idx_old[...] = c_old * L + posv
        idxr2[...] = lax.shift_right_logical(c_old * L + posv, pshift)
        cp_old = pltpu.async_copy(wdp_hbm.at[idxr2], wold_v, sem_w)
        cp_new.wait()
        cp_old.wait()
        co1 = pltpu.async_copy(wnew_v, outn_hbm.at[wid], sem_io)
        co2 = pltpu.async_copy(wold_v, outo_hbm.at[wid], sem_io)
        co3 = pltpu.async_copy(idx_new, outi_hbm.at[wid], sem_io)
        co4 = pltpu.async_copy(idx_old, outj_hbm.at[wid], sem_io)
        co1.wait()
        co2.wait()
        co3.wait()
        co4.wait()

    mesh = plsc.VectorSubcoreMesh(core_axis_name="c", subcore_axis_name="s",
                                  num_cores=2, num_subcores=16)
    return pl.kernel(
        body,
        out_type=(jax.ShapeDtypeStruct((B, _LANES, _GRAN), jnp.float32),
                  jax.ShapeDtypeStruct((B, _LANES, _GRAN), jnp.float32),
                  jax.ShapeDtypeStruct((B, _LANES), jnp.int32),
                  jax.ShapeDtypeStruct((B, _LANES), jnp.int32)),
        mesh=mesh,
        scratch_types=[
            pltpu.VMEM((_LANES,), jnp.int32),
            pltpu.VMEM((_LANES,), jnp.int32),
            pltpu.VMEM((_LANES,), jnp.int32),
            pltpu.VMEM((_LANES,), jnp.int32),
            pltpu.VMEM((_LANES,), jnp.int32),
            pltpu.VMEM((_LANES,), jnp.int32),
            pltpu.VMEM((_LANES,), jnp.int32),
            pltpu.VMEM((_LANES,), jnp.int32),
            pltpu.VMEM((_LANES,), jnp.int32),
            pltpu.VMEM((_LANES,), jnp.float32),
            pltpu.VMEM((_LANES,), jnp.float32),
            pltpu.VMEM((_LANES,), jnp.float32),
            pltpu.VMEM((_LANES, _GRAN), jnp.float32),
            pltpu.VMEM((_LANES, _GRAN), jnp.float32),
            pltpu.SemaphoreType.DMA,
            pltpu.SemaphoreType.DMA,
            pltpu.SemaphoreType.DMA,
        ],
    )


def _make_tc_body(B, C, d):
    def _tc_body(pn_ref, po_ref, ixn_ref, ixo_ref,
                 s0_ref, s1_ref, s2_ref, s3_ref, wd_ref, out_ref, acc_ref):
        k = pl.program_id(0)

        @pl.when(k == 0)
        def _init():
            acc_ref[...] = jnp.zeros_like(acc_ref)

        s_refs = (s0_ref, s1_ref, s2_ref, s3_ref)
        acc = None
        for c in range(C):
            part = jnp.dot(s_refs[c][...], wd_ref[c],
                           preferred_element_type=jnp.float32)
            acc = part if acc is None else acc + part
        acc_ref[...] += acc

        @pl.when(k == pl.num_programs(0) - 1)
        def _finish():
            u = acc_ref[...]
            pn = pn_ref[...]
            po = po_ref[...]
            sn = (ixn_ref[...] & 1)[:, :, None] == 1
            so = (ixo_ref[...] & 1)[:, :, None] == 1
            w_new = jnp.where(sn, pn[:, :, d:], pn[:, :, :d])
            w_old = jnp.where(so, po[:, :, d:], po[:, :, :d])
            v = u[:, None, :] + w_new - w_old
            nu = jnp.maximum(jnp.sqrt(jnp.sum(u * u, axis=-1)), 1e-12)
            nv = jnp.maximum(jnp.sqrt(jnp.sum(v * v, axis=-1)), 1e-12)
            uv = jnp.sum(u[:, None, :] * v, axis=-1)
            cos = jnp.minimum(uv / (nu[:, None] * nv), _MAXCOS)
            lt = -jnp.log(1.0 - cos)
            mask = lax.broadcasted_iota(jnp.int32, lt.shape, 1) < _NMUT
            out_ref[0, 0] = jnp.sum(jnp.where(mask, lt, 0.0)) / (lt.shape[0] * _NMUT)

    return _tc_body


def kernel(sequences, W, positions, nucleotides):
    B, L, C = sequences.shape
    d = W.shape[1]
    n = _NMUT
    scb = jnp.transpose(sequences, (2, 0, 1))
    scb2 = scb.reshape(C * B, L)
    seqf = scb.reshape(C * B * L)
    Wd = jnp.transpose(W.reshape(L, C, d), (1, 0, 2))
    wdp = Wd.reshape((C * L * d) // _GRAN, _GRAN)
    pos2 = jnp.zeros((B, _LANES), jnp.int32).at[:, :n].set(
        positions.reshape(B, n).astype(jnp.int32))
    nuc2 = jnp.zeros((B, _LANES), jnp.int32).at[:, :n].set(
        nucleotides.reshape(B, n).astype(jnp.int32))

    pn, po, ixn, ixo = _build_sc_gather(B, L, C, d)(wdp, seqf, pos2, nuc2)

    nk = L // _KB
    seq_specs = [pl.BlockSpec((B, _KB), (lambda k, c=c: (c, k)))
                 for c in range(C)]
    loss = pl.pallas_call(
        _make_tc_body(B, C, d),
        grid=(nk,),
        in_specs=[
            pl.BlockSpec((B, _LANES, _GRAN), lambda k: (0, 0, 0)),
            pl.BlockSpec((B, _LANES, _GRAN), lambda k: (0, 0, 0)),
            pl.BlockSpec((B, _LANES), lambda k: (0, 0)),
            pl.BlockSpec((B, _LANES), lambda k: (0, 0)),
            *seq_specs,
            pl.BlockSpec((C, _KB, d), lambda k: (0, k, 0)),
        ],
        out_specs=pl.BlockSpec((1, 1), lambda k: (0, 0),
                               memory_space=pltpu.SMEM),
        out_shape=jax.ShapeDtypeStruct((1, 1), jnp.float32),
        scratch_shapes=[pltpu.VMEM((B, d), jnp.float32)],
    )(pn, po, ixn, ixo, scb2, scb2, scb2, scb2, Wd)
    return loss[0, 0]


# speedup vs baseline: 1.0037x; 1.0037x over previous
import functools

import jax
import jax.numpy as jnp
from jax import lax
from jax.experimental import pallas as pl
from jax.experimental.pallas import tpu as pltpu
from jax.experimental.pallas import tpu_sc as plsc

_NMUT = 5
_MAXCOS = 0.99
_LANES = 16
_GRAN = 128
_KB = 2048


def _build_sc_gather(B, L, C, d):
    npair = _GRAN // d
    pshift = npair.bit_length() - 1
    assert npair == (1 << pshift) and d * npair == _GRAN and L % npair == 0

    def body(wdp_hbm, seqf_hbm, pos_hbm, nuc_hbm,
             outn_hbm, outo_hbm, outi_hbm, outj_hbm,
             pos_v, nuc_v, idx_old, idx_new, idxr1, idxr2,
             idx1, idx2, idx3, g1, g2, g3,
             wold_v, wnew_v, sem_g, sem_w, sem_io):
        cid = lax.axis_index("c")
        sid = lax.axis_index("s")
        wid = sid * 2 + cid
        cp_p = pltpu.async_copy(pos_hbm.at[wid], pos_v, sem_io)
        cp_n = pltpu.async_copy(nuc_hbm.at[wid], nuc_v, sem_io)
        cp_p.wait()
        cp_n.wait()
        posv = pos_v[...]
        nucv = nuc_v[...]
        idx_new[...] = nucv * L + posv
        idxr1[...] = lax.shift_right_logical(nucv * L + posv, pshift)
        cp_new = pltpu.async_copy(wdp_hbm.at[idxr1], wnew_v, sem_w)
        base = jnp.broadcast_to(wid * L, (_LANES,)).astype(jnp.int32) + posv
        idx1[...] = base + (B * L)
        cp1 = pltpu.async_copy(seqf_hbm.at[idx1], g1, sem_g)
        idx2[...] = base + 2 * (B * L)
        cp2 = pltpu.async_copy(seqf_hbm.at[idx2], g2, sem_g)
        idx3[...] = base + 3 * (B * L)
        cp3 = pltpu.async_copy(seqf_hbm.at[idx3], g3, sem_g)
        cp1.wait()
        cp2.wait()
        cp3.wait()
        c_old = (g1[...] + 2.0 * g2[...] + 3.0 * g3[...]).astype(jnp.int32)
        idx_old[...] = c_old * L + posv
        idxr2[...] = lax.shift_right_logical(c_old * L + posv, pshift)
        cp_old = pltpu.async_copy(wdp_hbm.at[idxr2], wold_v, sem_w)
        cp_new.wait()
        cp_old.wait()
        co1 = pltpu.async_copy(wnew_v, outn_hbm.at[wid], sem_io)
        co2 = pltpu.async_copy(wold_v, outo_hbm.at[wid], sem_io)
        co3 = pltpu.async_copy(idx_new, outi_hbm.at[wid], sem_io)
        co4 = pltpu.async_copy(idx_old, outj_hbm.at[wid], sem_io)
        co1.wait()
        co2.wait()
        co3.wait()
        co4.wait()

    mesh = plsc.VectorSubcoreMesh(core_axis_name="c", subcore_axis_name="s",
                                  num_cores=2, num_subcores=16)
    return pl.kernel(
        body,
        out_type=(jax.ShapeDtypeStruct((B, _LANES, _GRAN), jnp.float32),
                  jax.ShapeDtypeStruct((B, _LANES, _GRAN), jnp.float32),
                  jax.ShapeDtypeStruct((B, _LANES), jnp.int32),
                  jax.ShapeDtypeStruct((B, _LANES), jnp.int32)),
        mesh=mesh,
        scratch_types=[
            pltpu.VMEM((_LANES,), jnp.int32),
            pltpu.VMEM((_LANES,), jnp.int32),
            pltpu.VMEM((_LANES,), jnp.int32),
            pltpu.VMEM((_LANES,), jnp.int32),
            pltpu.VMEM((_LANES,), jnp.int32),
            pltpu.VMEM((_LANES,), jnp.int32),
            pltpu.VMEM((_LANES,), jnp.int32),
            pltpu.VMEM((_LANES,), jnp.int32),
            pltpu.VMEM((_LANES,), jnp.int32),
            pltpu.VMEM((_LANES,), jnp.float32),
            pltpu.VMEM((_LANES,), jnp.float32),
            pltpu.VMEM((_LANES,), jnp.float32),
            pltpu.VMEM((_LANES, _GRAN), jnp.float32),
            pltpu.VMEM((_LANES, _GRAN), jnp.float32),
            pltpu.SemaphoreType.DMA,
            pltpu.SemaphoreType.DMA,
            pltpu.SemaphoreType.DMA,
        ],
    )


def _make_tc_body(B, C, d):
    def _tc_body(pn_ref, po_ref, ixn_ref, ixo_ref,
                 s0_ref, s1_ref, s2_ref, s3_ref, wd_ref, out_ref, acc_ref):
        k = pl.program_id(0)

        @pl.when(k == 0)
        def _init():
            acc_ref[...] = jnp.zeros_like(acc_ref)

        s_refs = (s0_ref, s1_ref, s2_ref, s3_ref)
        acc = None
        for c in range(C):
            part = jnp.dot(s_refs[c][...], wd_ref[c],
                           preferred_element_type=jnp.float32)
            acc = part if acc is None else acc + part
        acc_ref[...] += acc

        @pl.when(k == pl.num_programs(0) - 1)
        def _finish():
            u = acc_ref[...]
            pn = pn_ref[...]
            po = po_ref[...]
            sn = (ixn_ref[...] & 1)[:, :, None] == 1
            so = (ixo_ref[...] & 1)[:, :, None] == 1
            w_new = jnp.where(sn, pn[:, :, d:], pn[:, :, :d])
            w_old = jnp.where(so, po[:, :, d:], po[:, :, :d])
            v = u[:, None, :] + w_new - w_old
            nu = jnp.maximum(jnp.sqrt(jnp.sum(u * u, axis=-1)), 1e-12)
            nv = jnp.maximum(jnp.sqrt(jnp.sum(v * v, axis=-1)), 1e-12)
            uv = jnp.sum(u[:, None, :] * v, axis=-1)
            cos = jnp.minimum(uv / (nu[:, None] * nv), _MAXCOS)
            lt = -jnp.log(1.0 - cos)
            mask = lax.broadcasted_iota(jnp.int32, lt.shape, 1) < _NMUT
            out_ref[0, 0] = jnp.sum(jnp.where(mask, lt, 0.0)) / (lt.shape[0] * _NMUT)

    return _tc_body


def kernel(sequences, W, positions, nucleotides):
    B, L, C = sequences.shape
    d = W.shape[1]
    n = _NMUT
    scb = jnp.transpose(sequences, (2, 0, 1))
    scb2 = scb.reshape(C * B, L)
    seqf = scb.reshape(C * B * L)
    Wd = jnp.transpose(W.T.reshape(d, L, C), (2, 1, 0))
    wdp = Wd.reshape((C * L * d) // _GRAN, _GRAN)
    pos2 = jnp.zeros((B, _LANES), jnp.int32).at[:, :n].set(
        positions.reshape(B, n).astype(jnp.int32))
    nuc2 = jnp.zeros((B, _LANES), jnp.int32).at[:, :n].set(
        nucleotides.reshape(B, n).astype(jnp.int32))

    pn, po, ixn, ixo = _build_sc_gather(B, L, C, d)(wdp, seqf, pos2, nuc2)

    nk = L // _KB
    seq_specs = [pl.BlockSpec((B, _KB), (lambda k, c=c: (c, k)))
                 for c in range(C)]
    loss = pl.pallas_call(
        _make_tc_body(B, C, d),
        grid=(nk,),
        in_specs=[
            pl.BlockSpec((B, _LANES, _GRAN), lambda k: (0, 0, 0)),
            pl.BlockSpec((B, _LANES, _GRAN), lambda k: (0, 0, 0)),
            pl.BlockSpec((B, _LANES), lambda k: (0, 0)),
            pl.BlockSpec((B, _LANES), lambda k: (0, 0)),
            *seq_specs,
            pl.BlockSpec((C, _KB, d), lambda k: (0, k, 0)),
        ],
        out_specs=pl.BlockSpec((1, 1), lambda k: (0, 0),
                               memory_space=pltpu.SMEM),
        out_shape=jax.ShapeDtypeStruct((1, 1), jnp.float32),
        scratch_shapes=[pltpu.VMEM((B, d), jnp.float32)],
    )(pn, po, ixn, ixo, scb2, scb2, scb2, scb2, Wd)
    return loss[0, 0]


# speedup vs baseline: 1.1021x; 1.0981x over previous
import functools

import jax
import jax.numpy as jnp
from jax import lax
from jax.experimental import pallas as pl
from jax.experimental.pallas import tpu as pltpu
from jax.experimental.pallas import tpu_sc as plsc

_NMUT = 5
_MAXCOS = 0.99
_LANES = 16
_GRAN = 128
_KB = 2048


def _build_sc_gather(B, L, C, d):
    npair = _GRAN // d
    pshift = npair.bit_length() - 1
    assert npair == (1 << pshift) and d * npair == _GRAN and L % npair == 0

    def body(wdp_hbm, seqf_hbm, pos_hbm, nuc_hbm,
             outn_hbm, outo_hbm, outi_hbm, outj_hbm,
             pos_v, nuc_v, idx_old, idx_new, idxr1, idxr2,
             idx1, idx2, idx3, g1, g2, g3,
             wold_v, wnew_v, sem_g, sem_w, sem_io):
        cid = lax.axis_index("c")
        sid = lax.axis_index("s")
        wid = sid * 2 + cid
        cp_p = pltpu.async_copy(pos_hbm.at[wid], pos_v, sem_io)
        cp_n = pltpu.async_copy(nuc_hbm.at[wid], nuc_v, sem_io)
        cp_p.wait()
        cp_n.wait()
        posv = pos_v[...]
        nucv = nuc_v[...]
        idx_new[...] = nucv * L + posv
        idxr1[...] = lax.shift_right_logical(nucv * L + posv, pshift)
        cp_new = pltpu.async_copy(wdp_hbm.at[idxr1], wnew_v, sem_w)
        base = jnp.broadcast_to(wid * (C * L), (_LANES,)).astype(jnp.int32) + posv
        idx1[...] = base + L
        cp1 = pltpu.async_copy(seqf_hbm.at[idx1], g1, sem_g)
        idx2[...] = base + 2 * L
        cp2 = pltpu.async_copy(seqf_hbm.at[idx2], g2, sem_g)
        idx3[...] = base + 3 * L
        cp3 = pltpu.async_copy(seqf_hbm.at[idx3], g3, sem_g)
        cp1.wait()
        cp2.wait()
        cp3.wait()
        c_old = (g1[...] + 2.0 * g2[...] + 3.0 * g3[...]).astype(jnp.int32)
        idx_old[...] = c_old * L + posv
        idxr2[...] = lax.shift_right_logical(c_old * L + posv, pshift)
        cp_old = pltpu.async_copy(wdp_hbm.at[idxr2], wold_v, sem_w)
        cp_new.wait()
        cp_old.wait()
        co1 = pltpu.async_copy(wnew_v, outn_hbm.at[wid], sem_io)
        co2 = pltpu.async_copy(wold_v, outo_hbm.at[wid], sem_io)
        co3 = pltpu.async_copy(idx_new, outi_hbm.at[wid], sem_io)
        co4 = pltpu.async_copy(idx_old, outj_hbm.at[wid], sem_io)
        co1.wait()
        co2.wait()
        co3.wait()
        co4.wait()

    mesh = plsc.VectorSubcoreMesh(core_axis_name="c", subcore_axis_name="s",
                                  num_cores=2, num_subcores=16)
    return pl.kernel(
        body,
        out_type=(jax.ShapeDtypeStruct((B, _LANES, _GRAN), jnp.float32),
                  jax.ShapeDtypeStruct((B, _LANES, _GRAN), jnp.float32),
                  jax.ShapeDtypeStruct((B, _LANES), jnp.int32),
                  jax.ShapeDtypeStruct((B, _LANES), jnp.int32)),
        mesh=mesh,
        scratch_types=[
            pltpu.VMEM((_LANES,), jnp.int32),
            pltpu.VMEM((_LANES,), jnp.int32),
            pltpu.VMEM((_LANES,), jnp.int32),
            pltpu.VMEM((_LANES,), jnp.int32),
            pltpu.VMEM((_LANES,), jnp.int32),
            pltpu.VMEM((_LANES,), jnp.int32),
            pltpu.VMEM((_LANES,), jnp.int32),
            pltpu.VMEM((_LANES,), jnp.int32),
            pltpu.VMEM((_LANES,), jnp.int32),
            pltpu.VMEM((_LANES,), jnp.float32),
            pltpu.VMEM((_LANES,), jnp.float32),
            pltpu.VMEM((_LANES,), jnp.float32),
            pltpu.VMEM((_LANES, _GRAN), jnp.float32),
            pltpu.VMEM((_LANES, _GRAN), jnp.float32),
            pltpu.SemaphoreType.DMA,
            pltpu.SemaphoreType.DMA,
            pltpu.SemaphoreType.DMA,
        ],
    )


def _make_tc_body(B, C, d):
    def _tc_body(pn_ref, po_ref, ixn_ref, ixo_ref,
                 seqt_ref, wd_ref, out_ref, acc_ref):
        k = pl.program_id(0)

        @pl.when(k == 0)
        def _init():
            acc_ref[...] = jnp.zeros_like(acc_ref)

        st = seqt_ref[...]
        acc = None
        for c in range(C):
            part = jnp.dot(st[:, c, :], wd_ref[c],
                           preferred_element_type=jnp.float32)
            acc = part if acc is None else acc + part
        acc_ref[...] += acc

        @pl.when(k == pl.num_programs(0) - 1)
        def _finish():
            u = acc_ref[...]
            pn = pn_ref[...]
            po = po_ref[...]
            sn = (ixn_ref[...] & 1)[:, :, None] == 1
            so = (ixo_ref[...] & 1)[:, :, None] == 1
            w_new = jnp.where(sn, pn[:, :, d:], pn[:, :, :d])
            w_old = jnp.where(so, po[:, :, d:], po[:, :, :d])
            v = u[:, None, :] + w_new - w_old
            nu = jnp.maximum(jnp.sqrt(jnp.sum(u * u, axis=-1)), 1e-12)
            nv = jnp.maximum(jnp.sqrt(jnp.sum(v * v, axis=-1)), 1e-12)
            uv = jnp.sum(u[:, None, :] * v, axis=-1)
            cos = jnp.minimum(uv / (nu[:, None] * nv), _MAXCOS)
            lt = -jnp.log(1.0 - cos)
            mask = lax.broadcasted_iota(jnp.int32, lt.shape, 1) < _NMUT
            out_ref[0, 0] = jnp.sum(jnp.where(mask, lt, 0.0)) / (lt.shape[0] * _NMUT)

    return _tc_body


def kernel(sequences, W, positions, nucleotides):
    B, L, C = sequences.shape
    d = W.shape[1]
    n = _NMUT
    seqT = jnp.transpose(sequences, (0, 2, 1))
    seqf = seqT.reshape(B * C * L)
    Wd = jnp.transpose(W.T.reshape(d, L, C), (2, 1, 0))
    wdp = Wd.reshape((C * L * d) // _GRAN, _GRAN)
    pos2 = jnp.zeros((B, _LANES), jnp.int32).at[:, :n].set(
        positions.reshape(B, n).astype(jnp.int32))
    nuc2 = jnp.zeros((B, _LANES), jnp.int32).at[:, :n].set(
        nucleotides.reshape(B, n).astype(jnp.int32))

    pn, po, ixn, ixo = _build_sc_gather(B, L, C, d)(wdp, seqf, pos2, nuc2)

    nk = L // _KB
    loss = pl.pallas_call(
        _make_tc_body(B, C, d),
        grid=(nk,),
        in_specs=[
            pl.BlockSpec((B, _LANES, _GRAN), lambda k: (0, 0, 0)),
            pl.BlockSpec((B, _LANES, _GRAN), lambda k: (0, 0, 0)),
            pl.BlockSpec((B, _LANES), lambda k: (0, 0)),
            pl.BlockSpec((B, _LANES), lambda k: (0, 0)),
            pl.BlockSpec((B, C, _KB), lambda k: (0, 0, k)),
            pl.BlockSpec((C, _KB, d), lambda k: (0, k, 0)),
        ],
        out_specs=pl.BlockSpec((1, 1), lambda k: (0, 0),
                               memory_space=pltpu.SMEM),
        out_shape=jax.ShapeDtypeStruct((1, 1), jnp.float32),
        scratch_shapes=[pltpu.VMEM((B, d), jnp.float32)],
    )(pn, po, ixn, ixo, seqT, Wd)
    return loss[0, 0]


# speedup vs baseline: 1.1269x; 1.0225x over previous
import functools

import jax
import jax.numpy as jnp
from jax import lax
from jax.experimental import pallas as pl
from jax.experimental.pallas import tpu as pltpu
from jax.experimental.pallas import tpu_sc as plsc

_NMUT = 5
_MAXCOS = 0.99
_LANES = 16
_GRAN = 128
_KB = 2048


def _build_sc_gather(B, L, C, d):
    def body(wp2_hbm, seqf_hbm, pos_hbm, nuc_hbm,
             outn_hbm, outo_hbm,
             pos_v, nuc_v, idx_old, idx_new,
             idx1, idx2, idx3, g1, g2, g3,
             wold_v, wnew_v, sem_g, sem_w, sem_io):
        cid = lax.axis_index("c")
        sid = lax.axis_index("s")
        wid = sid * 2 + cid
        cp_p = pltpu.async_copy(pos_hbm.at[wid], pos_v, sem_io)
        cp_n = pltpu.async_copy(nuc_hbm.at[wid], nuc_v, sem_io)
        cp_p.wait()
        cp_n.wait()
        posv = pos_v[...]
        nucv = nuc_v[...]
        idx_new[...] = nucv * L + posv
        cp_new = pltpu.async_copy(wp2_hbm.at[idx_new], wnew_v, sem_w)
        base = jnp.broadcast_to(wid * (C * L), (_LANES,)).astype(jnp.int32) + posv
        idx1[...] = base + L
        cp1 = pltpu.async_copy(seqf_hbm.at[idx1], g1, sem_g)
        idx2[...] = base + 2 * L
        cp2 = pltpu.async_copy(seqf_hbm.at[idx2], g2, sem_g)
        idx3[...] = base + 3 * L
        cp3 = pltpu.async_copy(seqf_hbm.at[idx3], g3, sem_g)
        cp1.wait()
        cp2.wait()
        cp3.wait()
        c_old = (g1[...] + 2.0 * g2[...] + 3.0 * g3[...]).astype(jnp.int32)
        idx_old[...] = c_old * L + posv
        cp_old = pltpu.async_copy(wp2_hbm.at[idx_old], wold_v, sem_w)
        cp_new.wait()
        cp_old.wait()
        co1 = pltpu.async_copy(wnew_v, outn_hbm.at[wid], sem_io)
        co2 = pltpu.async_copy(wold_v, outo_hbm.at[wid], sem_io)
        co1.wait()
        co2.wait()

    mesh = plsc.VectorSubcoreMesh(core_axis_name="c", subcore_axis_name="s",
                                  num_cores=2, num_subcores=16)
    return pl.kernel(
        body,
        out_type=(jax.ShapeDtypeStruct((B, _LANES, _GRAN), jnp.float32),
                  jax.ShapeDtypeStruct((B, _LANES, _GRAN), jnp.float32)),
        mesh=mesh,
        scratch_types=[
            pltpu.VMEM((_LANES,), jnp.int32),
            pltpu.VMEM((_LANES,), jnp.int32),
            pltpu.VMEM((_LANES,), jnp.int32),
            pltpu.VMEM((_LANES,), jnp.int32),
            pltpu.VMEM((_LANES,), jnp.int32),
            pltpu.VMEM((_LANES,), jnp.int32),
            pltpu.VMEM((_LANES,), jnp.int32),
            pltpu.VMEM((_LANES,), jnp.float32),
            pltpu.VMEM((_LANES,), jnp.float32),
            pltpu.VMEM((_LANES,), jnp.float32),
            pltpu.VMEM((_LANES, _GRAN), jnp.float32),
            pltpu.VMEM((_LANES, _GRAN), jnp.float32),
            pltpu.SemaphoreType.DMA,
            pltpu.SemaphoreType.DMA,
            pltpu.SemaphoreType.DMA,
        ],
    )


def _make_tc_body(B, C, d):
    def _tc_body(pn_ref, po_ref, seqt_ref, wp_ref, out_ref, acc_ref):
        k = pl.program_id(0)

        @pl.when(k == 0)
        def _init():
            acc_ref[...] = jnp.zeros_like(acc_ref)

        st = seqt_ref[...]
        acc = None
        for c in range(C):
            part = jnp.dot(st[:, c, :], wp_ref[c][:, :d],
                           preferred_element_type=jnp.float32)
            acc = part if acc is None else acc + part
        acc_ref[...] += acc

        @pl.when(k == pl.num_programs(0) - 1)
        def _finish():
            u = acc_ref[...]
            w_new = pn_ref[:, :, :d]
            w_old = po_ref[:, :, :d]
            v = u[:, None, :] + w_new - w_old
            nu = jnp.maximum(jnp.sqrt(jnp.sum(u * u, axis=-1)), 1e-12)
            nv = jnp.maximum(jnp.sqrt(jnp.sum(v * v, axis=-1)), 1e-12)
            uv = jnp.sum(u[:, None, :] * v, axis=-1)
            cos = jnp.minimum(uv / (nu[:, None] * nv), _MAXCOS)
            lt = -jnp.log(1.0 - cos)
            mask = lax.broadcasted_iota(jnp.int32, lt.shape, 1) < _NMUT
            out_ref[0, 0] = jnp.sum(jnp.where(mask, lt, 0.0)) / (lt.shape[0] * _NMUT)

    return _tc_body


def kernel(sequences, W, positions, nucleotides):
    B, L, C = sequences.shape
    d = W.shape[1]
    n = _NMUT
    seqT = jnp.transpose(sequences, (0, 2, 1))
    seqf = seqT.reshape(B * C * L)
    Wp = jnp.pad(jnp.transpose(W.T.reshape(d, L, C), (2, 1, 0)),
                 ((0, 0), (0, 0), (0, _GRAN - d)))
    wp2 = Wp.reshape(C * L, _GRAN)
    pos2 = jnp.zeros((B, _LANES), jnp.int32).at[:, :n].set(
        positions.reshape(B, n).astype(jnp.int32))
    nuc2 = jnp.zeros((B, _LANES), jnp.int32).at[:, :n].set(
        nucleotides.reshape(B, n).astype(jnp.int32))

    pn, po = _build_sc_gather(B, L, C, d)(wp2, seqf, pos2, nuc2)

    nk = L // _KB
    loss = pl.pallas_call(
        _make_tc_body(B, C, d),
        grid=(nk,),
        in_specs=[
            pl.BlockSpec((B, _LANES, _GRAN), lambda k: (0, 0, 0)),
            pl.BlockSpec((B, _LANES, _GRAN), lambda k: (0, 0, 0)),
            pl.BlockSpec((B, C, _KB), lambda k: (0, 0, k)),
            pl.BlockSpec((C, _KB, _GRAN), lambda k: (0, k, 0)),
        ],
        out_specs=pl.BlockSpec((1, 1), lambda k: (0, 0),
                               memory_space=pltpu.SMEM),
        out_shape=jax.ShapeDtypeStruct((1, 1), jnp.float32),
        scratch_shapes=[pltpu.VMEM((B, d), jnp.float32)],
    )(pn, po, seqT, Wp)
    return loss[0, 0]
